# baseline (speedup 1.0000x reference)
import functools

import jax
import jax.numpy as jnp
from jax.experimental import pallas as pl
from jax.experimental.pallas import tpu as pltpu
from jax.experimental.pallas import tpu_sc as plsc

NUM_EXPERTS = 64
TOP_K = 8
HIDDEN = 1024
INTER = 512
BATCH = 128

_NEG = -3e38
_NUM_WORKERS = 32
_TOK_PER_W = BATCH // _NUM_WORKERS
_VREGS = NUM_EXPERTS // 16
_PRE = 8
_DUMP = _TOK_PER_W * NUM_EXPERTS


_GDN = jax.lax.GatherDimensionNumbers(
    offset_dims=(), collapsed_slice_dims=(0,), start_index_map=(0,))


def _shuf(x, idx):
    return jax.lax.gather(x, idx[:, None], _GDN, slice_sizes=(1,),
                          mode=jax.lax.GatherScatterMode.PROMISE_IN_BOUNDS)


def _bcast_red(x, op, lane):
    for sh in (1, 2, 4, 8):
        x = op(x, _shuf(x, lane ^ sh))
    return x


def _routing_body(logits_hbm, comb_hbm, lg_v, comb_v):
    wid = jax.lax.axis_index("s") * 2 + jax.lax.axis_index("c")
    base = wid * _TOK_PER_W
    pltpu.sync_copy(logits_hbm.at[pl.ds(base, _TOK_PER_W)], lg_v)

    lane = jax.lax.broadcasted_iota(jnp.int32, (16,), 0)
    for r in range(_TOK_PER_W):
        v = [lg_v[r, pl.ds(j * 16, 16)] for j in range(_VREGS)]
        picks = []
        for k in range(TOP_K):
            m = jnp.maximum(jnp.maximum(v[0], v[1]), jnp.maximum(v[2], v[3]))
            s = _bcast_red(m, jnp.maximum, lane)
            idxv = jnp.where(v[0] == s, lane, NUM_EXPERTS)
            for j in range(1, _VREGS):
                idxv = jnp.minimum(idxv, jnp.where(v[j] == s, lane + j * 16, NUM_EXPERTS))
            first = _bcast_red(idxv, jnp.minimum, lane)
            picks.append((first, s))
            j_sel = jax.lax.shift_right_logical(first, 4)
            l_sel = jax.lax.bitwise_and(first, 15)
            for j in range(_VREGS):
                v[j] = jnp.where((lane == l_sel) & (j_sel == j), _NEG, v[j])
        s0 = picks[0][1]
        exps = [jnp.exp(s - s0) for _, s in picks]
        denom = exps[0]
        for t in exps[1:]:
            denom = denom + t
        inv = 1.0 / denom
        for j in range(_VREGS):
            chunk = jnp.zeros((16,), jnp.float32)
            lane_j = lane + j * 16
            for k in range(TOP_K):
                chunk = chunk + jnp.where(lane_j == picks[k][0], exps[k] * inv, 0.0)
            comb_v[pl.ds(r * NUM_EXPERTS + j * 16, 16)] = chunk

    pltpu.sync_copy(comb_v, comb_hbm.at[pl.ds(base * NUM_EXPERTS, _TOK_PER_W * NUM_EXPERTS)])


_route = functools.partial(
    pl.kernel,
    mesh=plsc.VectorSubcoreMesh(core_axis_name="c", subcore_axis_name="s"),
    out_type=jax.ShapeDtypeStruct((BATCH * NUM_EXPERTS,), jnp.float32),
    scratch_types=[
        pltpu.VMEM((_TOK_PER_W, NUM_EXPERTS), jnp.float32),
        pltpu.VMEM((_TOK_PER_W * NUM_EXPERTS,), jnp.float32),
    ],
)(_routing_body)


def _pre_body(xT_ref, wg_ref, wu_ref, wd_ref, y8_ref):
    xT = xT_ref[...]
    dn = (((1,), (0,)), ((), ()))
    g = jax.lax.dot_general(wg_ref[0], xT, dn, preferred_element_type=jnp.float32)
    u = jax.lax.dot_general(wu_ref[0], xT, dn, preferred_element_type=jnp.float32)
    h = (g * jax.nn.sigmoid(g)) * u
    y = jax.lax.dot_general(wd_ref[0], h, dn, preferred_element_type=jnp.float32)
    y8_ref[0] = y.astype(jnp.bfloat16)


def _moe_body(comb_ref, xT_ref, y8_ref, wg_ref, wu_ref, wd_ref, outT_ref, combT_ref):
    e = pl.program_id(0)

    @pl.when(e == 0)
    def _transpose_comb():
        combT_ref[...] = comb_ref[...].T

    def _apply(y):
        c = combT_ref[pl.ds(e, 1), :]

        @pl.when(e == 0)
        def _init():
            outT_ref[...] = y * c

        @pl.when(e > 0)
        def _acc():
            outT_ref[...] += y * c

    @pl.when(e < _PRE)
    def _use_pre():
        _apply(y8_ref[0].astype(jnp.float32))

    @pl.when(e >= _PRE)
    def _compute():
        xT = xT_ref[...]
        dn = (((1,), (0,)), ((), ()))
        g = jax.lax.dot_general(wg_ref[0], xT, dn, preferred_element_type=jnp.float32)
        u = jax.lax.dot_general(wu_ref[0], xT, dn, preferred_element_type=jnp.float32)
        h = (g * jax.nn.sigmoid(g)) * u
        _apply(jax.lax.dot_general(wd_ref[0], h, dn, preferred_element_type=jnp.float32))


def kernel(x, gate_w, w_gate, w_up, w_down):
    if x.ndim == 3:
        x2 = x[:, -1, :]
    else:
        x2 = x
    logits = (x2.astype(jnp.float16) @ gate_w.T.astype(jnp.float16)).astype(x2.dtype)
    xT = x2.T

    y8 = pl.pallas_call(
        _pre_body,
        grid=(_PRE,),
        in_specs=[
            pl.BlockSpec((HIDDEN, BATCH), lambda e: (0, 0)),
            pl.BlockSpec((1, INTER, HIDDEN), lambda e: (e, 0, 0)),
            pl.BlockSpec((1, INTER, HIDDEN), lambda e: (e, 0, 0)),
            pl.BlockSpec((1, HIDDEN, INTER), lambda e: (e, 0, 0)),
        ],
        out_specs=pl.BlockSpec((1, HIDDEN, BATCH), lambda e: (e, 0, 0)),
        out_shape=jax.ShapeDtypeStruct((_PRE, HIDDEN, BATCH), jnp.bfloat16),
        compiler_params=pltpu.CompilerParams(
            dimension_semantics=("arbitrary",),
        ),
    )(xT, w_gate, w_up, w_down)

    comb = _route(logits).reshape(BATCH, NUM_EXPERTS)

    outT = pl.pallas_call(
        _moe_body,
        grid=(NUM_EXPERTS,),
        in_specs=[
            pl.BlockSpec((BATCH, NUM_EXPERTS), lambda e: (0, 0)),
            pl.BlockSpec((HIDDEN, BATCH), lambda e: (0, 0)),
            pl.BlockSpec((1, HIDDEN, BATCH), lambda e: (jnp.minimum(e, _PRE - 1), 0, 0)),
            pl.BlockSpec((1, INTER, HIDDEN), lambda e: (jnp.maximum(e, _PRE), 0, 0)),
            pl.BlockSpec((1, INTER, HIDDEN), lambda e: (jnp.maximum(e, _PRE), 0, 0)),
            pl.BlockSpec((1, HIDDEN, INTER), lambda e: (jnp.maximum(e, _PRE), 0, 0)),
        ],
        out_specs=pl.BlockSpec((HIDDEN, BATCH), lambda e: (0, 0)),
        out_shape=jax.ShapeDtypeStruct((HIDDEN, BATCH), jnp.float32),
        scratch_shapes=[pltpu.VMEM((NUM_EXPERTS, BATCH), jnp.float32)],
        compiler_params=pltpu.CompilerParams(
            dimension_semantics=("arbitrary",),
        ),
    )(comb, xT, y8, w_gate, w_up, w_down)
    return outT.T


# speedup vs baseline: 1.1712x; 1.1712x over previous
import jax
import jax.numpy as jnp
from jax.experimental import pallas as pl
from jax.experimental.pallas import tpu as pltpu

NUM_EXPERTS = 64
TOP_K = 8
HIDDEN = 1024
INTER = 512
BATCH = 128

_NEG = -3e38


def _moe_body(logits_ref, x_ref, wg_ref, wu_ref, wd_ref, out_ref,
              comb_ref, xT_ref, outT_ref):
    e = pl.program_id(0)

    @pl.when(e == 0)
    def _prologue():
        xT_ref[...] = x_ref[...].T
        lg = logits_ref[...]
        ids = jax.lax.broadcasted_iota(jnp.int32, (BATCH, NUM_EXPERTS), 1)
        work = lg
        vals = []
        sels = []
        for _ in range(TOP_K):
            m = jnp.max(work, axis=1, keepdims=True)
            is_m = work == m
            first = jnp.min(jnp.where(is_m, ids, NUM_EXPERTS), axis=1, keepdims=True)
            sel = ids == first
            vals.append(m)
            sels.append(sel)
            work = jnp.where(sel, _NEG, work)
        exps = [jnp.exp(v - vals[0]) for v in vals]
        denom = exps[0]
        for t in exps[1:]:
            denom = denom + t
        comb = jnp.zeros((BATCH, NUM_EXPERTS), jnp.float32)
        for k in range(TOP_K):
            comb = comb + sels[k].astype(jnp.float32) * (exps[k] / denom)
        comb_ref[...] = comb.T

    xT = xT_ref[...]
    wg = wg_ref[0]
    wu = wu_ref[0]
    wd = wd_ref[0]
    dn = (((1,), (0,)), ((), ()))
    g = jax.lax.dot_general(wg, xT, dn, preferred_element_type=jnp.float32)
    u = jax.lax.dot_general(wu, xT, dn, preferred_element_type=jnp.float32)
    h = (g * jax.nn.sigmoid(g)) * u
    y = jax.lax.dot_general(wd, h, dn, preferred_element_type=jnp.float32)
    c = comb_ref[pl.ds(e, 1), :]

    @pl.when(e == 0)
    def _init():
        outT_ref[...] = y * c

    @pl.when(e > 0)
    def _acc():
        outT_ref[...] += y * c

    @pl.when(e == NUM_EXPERTS - 1)
    def _epilogue():
        out_ref[...] = outT_ref[...].T


def kernel(x, gate_w, w_gate, w_up, w_down):
    if x.ndim == 3:
        x2 = x[:, -1, :]
    else:
        x2 = x
    logits = (x2.astype(jnp.float16) @ gate_w.T.astype(jnp.float16)).astype(x2.dtype)

    out = pl.pallas_call(
        _moe_body,
        grid=(NUM_EXPERTS,),
        in_specs=[
            pl.BlockSpec((BATCH, NUM_EXPERTS), lambda e: (0, 0)),
            pl.BlockSpec((BATCH, HIDDEN), lambda e: (0, 0)),
            pl.BlockSpec((1, INTER, HIDDEN), lambda e: (e, 0, 0)),
            pl.BlockSpec((1, INTER, HIDDEN), lambda e: (e, 0, 0)),
            pl.BlockSpec((1, HIDDEN, INTER), lambda e: (e, 0, 0)),
        ],
        out_specs=pl.BlockSpec((BATCH, HIDDEN), lambda e: (0, 0)),
        out_shape=jax.ShapeDtypeStruct((BATCH, HIDDEN), jnp.float32),
        scratch_shapes=[
            pltpu.VMEM((NUM_EXPERTS, BATCH), jnp.float32),
            pltpu.VMEM((HIDDEN, BATCH), jnp.float32),
            pltpu.VMEM((HIDDEN, BATCH), jnp.float32),
        ],
        compiler_params=pltpu.CompilerParams(
            dimension_semantics=("arbitrary",),
        ),
    )(logits, x2, w_gate, w_up, w_down)
    return out
